# baseline (speedup 1.0000x reference)
import functools

import jax
import jax.numpy as jnp
from jax import lax
from jax.experimental import pallas as pl
from jax.experimental.pallas import tpu as pltpu
from jax.experimental.pallas import tpu_sc as plsc

N = 10000
D = 128
D_OUT = 16
E = 80000
R = 4

NC = 2
NS = 16
LANES = 16

N_PAD = 10240
E_PAD = 81920
JUNK = 10200
CH = 128
ROWS_PER_TILE = N_PAD // NS
CHUNKS_PER_TILE = E_PAD // NS // CH
DEG_TOTAL = 4 * N_PAD
DEG_PER_TILE = DEG_TOTAL // NS

_sc_mesh = plsc.VectorSubcoreMesh(
    core_axis_name="c", subcore_axis_name="s", num_cores=NC, num_subcores=NS)


CH_E = 4096
N_CHUNKS_E = E_PAD // CH_E
SUB = N_PAD // NS


def _sc_degrees_body(s0, s1, s2, s3, d0, d1, d2, d3, deg_out,
                     idxbuf, subhist, histbuf):
  c = lax.axis_index("c")
  s = lax.axis_index("s")
  base = s * SUB
  lane_off = lax.broadcasted_iota(jnp.int32, (LANES,), 0) * SUB
  ones = jnp.ones((LANES,), jnp.float32)
  edge_arrays = ((s0, d0), (s1, d1), (s2, d2), (s3, d3))

  for rel in range(R):

    @pl.when(c == rel // 2)
    def _():
      for side in range(2):
        idx_hbm = edge_arrays[rel][side]
        a = 2 * rel + side

        def zero_blk(i, carry):
          subhist[pl.ds(i * LANES, LANES)] = jnp.zeros((LANES,), jnp.float32)
          return carry

        lax.fori_loop(0, (LANES * SUB) // LANES, zero_blk, 0)

        def chunk(k, carry):
          pltpu.sync_copy(idx_hbm.at[pl.ds(k * CH_E, CH_E)], idxbuf)

          def vec(j, carry2):
            idx = idxbuf[pl.ds(j * LANES, LANES)]
            rel_idx = idx - base
            m = (rel_idx >= 0) & (rel_idx < SUB)
            rel_c = jnp.clip(rel_idx, 0, SUB - 1)
            plsc.addupdate_scatter(subhist, [lane_off + rel_c], ones, mask=m)
            return carry2

          lax.fori_loop(0, CH_E // LANES, vec, 0)
          return carry

        lax.fori_loop(0, N_CHUNKS_E, chunk, 0)

        def reduce_blk(i, carry):
          acc = jnp.zeros((LANES,), jnp.float32)
          for lane in range(LANES):
            acc = acc + subhist[pl.ds(lane * SUB + i * LANES, LANES)]
          histbuf[pl.ds(i * LANES, LANES)] = acc
          return carry

        lax.fori_loop(0, SUB // LANES, reduce_blk, 0)
        pltpu.sync_copy(histbuf, deg_out.at[pl.ds(a * N_PAD + base, SUB)])


_sc_degrees = pl.kernel(
    _sc_degrees_body,
    out_type=jax.ShapeDtypeStruct((8 * N_PAD,), jnp.float32),
    mesh=_sc_mesh,
    compiler_params=pltpu.CompilerParams(needs_layout_passes=False),
    scratch_types=[
        pltpu.VMEM((CH_E,), jnp.int32),
        pltpu.VMEM((LANES * SUB,), jnp.float32),
        pltpu.VMEM((SUB,), jnp.float32),
    ],
)


def _sc_aggregate_body(y0, y1, y2, y3, s0, s1, s2, s3, d0, d1, d2, d3, zrows,
                       a0, a1, a2, a3, shacc, idx_s, idx_d, rows, sem):
  c = lax.axis_index("c")
  s = lax.axis_index("s")
  ys = (y0, y1, y2, y3)
  srcs = (s0, s1, s2, s3)
  dsts = (d0, d1, d2, d3)
  aggs = (a0, a1, a2, a3)
  row_sl = pl.ds(s * ROWS_PER_TILE, ROWS_PER_TILE)

  for rel in range(R):

    @pl.when(c == rel // 2)
    def _():
      y = ys[rel]
      src = srcs[rel]
      dst = dsts[rel]

      pltpu.sync_copy(zrows.at[row_sl], shacc.at[row_sl])
      plsc.subcore_barrier()

      def chunk(k, carry):
        off = s * (E_PAD // NS) + k * CH
        pltpu.sync_copy(src.at[pl.ds(off, CH)], idx_s)
        pltpu.sync_copy(dst.at[pl.ds(off, CH)], idx_d)
        pltpu.async_copy(y.at[idx_s], rows, sem).wait()
        pltpu.sync_copy(rows, shacc.at[idx_d], add=True)
        return carry

      lax.fori_loop(0, CHUNKS_PER_TILE, chunk, 0)
      plsc.subcore_barrier()
      pltpu.sync_copy(shacc.at[row_sl], aggs[rel].at[row_sl])
      plsc.subcore_barrier()


_sc_aggregate = pl.kernel(
    _sc_aggregate_body,
    out_type=[jax.ShapeDtypeStruct((N_PAD, D), jnp.float32)] * R,
    mesh=_sc_mesh,
    compiler_params=pltpu.CompilerParams(use_tc_tiling_on_sc=False),
    scratch_types=[
        pltpu.VMEM_SHARED((N_PAD, D), jnp.float32),
        pltpu.VMEM((CH,), jnp.int32),
        pltpu.VMEM((CH,), jnp.int32),
        pltpu.VMEM((CH, D), jnp.float32),
        pltpu.SemaphoreType.DMA,
    ],
)


_BLK = 512
_GRID = N_PAD // _BLK


def _norm(deg_row):
  return lax.rsqrt(jnp.maximum(deg_row, 1.0))


def _tc_layer1_body(x_ref, deg_ref, w_ref, y0, y1, y2, y3):
  xb = x_ref[...]
  outs = (y0, y1, y2, y3)
  for r in range(R):
    ns = _norm(deg_ref[2 * r])
    outs[r][...] = jnp.dot(xb * ns[:, None], w_ref[r],
                           preferred_element_type=jnp.float32)


_tc_layer1 = pl.pallas_call(
    _tc_layer1_body,
    grid=(_GRID,),
    in_specs=[
        pl.BlockSpec((_BLK, D), lambda i: (i, 0)),
        pl.BlockSpec((8, _BLK), lambda i: (0, i)),
        pl.BlockSpec((R, D, D), lambda i: (0, 0, 0)),
    ],
    out_specs=[pl.BlockSpec((_BLK, D), lambda i: (i, 0))] * R,
    out_shape=[jax.ShapeDtypeStruct((N_PAD, D), jnp.float32)] * R,
)


def _tc_layer2_body(a0, a1, a2, a3, deg_ref, b_ref, w_ref, y0, y1, y2, y3):
  aggs = (a0, a1, a2, a3)
  h = jnp.zeros((_BLK, D), jnp.float32)
  for r in range(R):
    nd = _norm(deg_ref[2 * r + 1])
    h = h + aggs[r][...] * nd[:, None] + b_ref[r][None, :]
  h = jnp.maximum(h, 0.0)
  outs = (y0, y1, y2, y3)
  for r in range(R):
    ns = _norm(deg_ref[2 * r])
    outs[r][...] = jnp.dot(h * ns[:, None], w_ref[r],
                           preferred_element_type=jnp.float32)


_tc_layer2 = pl.pallas_call(
    _tc_layer2_body,
    grid=(_GRID,),
    in_specs=[pl.BlockSpec((_BLK, D), lambda i: (i, 0))] * R + [
        pl.BlockSpec((8, _BLK), lambda i: (0, i)),
        pl.BlockSpec((R, D), lambda i: (0, 0)),
        pl.BlockSpec((R, D, D), lambda i: (0, 0, 0)),
    ],
    out_specs=[pl.BlockSpec((_BLK, D), lambda i: (i, 0))] * R,
    out_shape=[jax.ShapeDtypeStruct((N_PAD, D), jnp.float32)] * R,
)


def _tc_final_body(a0, a1, a2, a3, deg_ref, b_ref, wc_ref, bc_ref, out_ref):
  aggs = (a0, a1, a2, a3)
  h = jnp.zeros((_BLK, D), jnp.float32)
  for r in range(R):
    nd = _norm(deg_ref[2 * r + 1])
    h = h + aggs[r][...] * nd[:, None] + b_ref[r][None, :]
  h = jnp.maximum(h, 0.0)
  out_ref[...] = jnp.dot(h, wc_ref[...],
                         preferred_element_type=jnp.float32) + bc_ref[0][None, :]


_tc_final = pl.pallas_call(
    _tc_final_body,
    grid=(_GRID,),
    in_specs=[pl.BlockSpec((_BLK, D), lambda i: (i, 0))] * R + [
        pl.BlockSpec((8, _BLK), lambda i: (0, i)),
        pl.BlockSpec((R, D), lambda i: (0, 0)),
        pl.BlockSpec((D, D_OUT), lambda i: (0, 0)),
        pl.BlockSpec((1, D_OUT), lambda i: (0, 0)),
    ],
    out_specs=pl.BlockSpec((_BLK, D_OUT), lambda i: (i, 0)),
    out_shape=jax.ShapeDtypeStruct((N_PAD, D_OUT), jnp.float32),
)


@jax.jit
def kernel(x, W1_0, b1_0, W1_1, b1_1, W1_2, b1_2, W1_3, b1_3,
           W2_0, b2_0, W2_1, b2_1, W2_2, b2_2, W2_3, b2_3,
           Wc, bc, edge_index0, edge_index1, edge_index2, edge_index3):
  edges = (edge_index0, edge_index1, edge_index2, edge_index3)
  pad = jnp.full((E_PAD - E,), JUNK, dtype=jnp.int32)
  srcs = [jnp.concatenate([e[0], pad]) for e in edges]
  dsts = [jnp.concatenate([e[1], pad]) for e in edges]

  x_pad = jnp.zeros((N_PAD, D), jnp.float32).at[:N].set(x)
  zrows = jnp.zeros((N_PAD, D), jnp.float32)

  deg_flat = _sc_degrees(*srcs, *dsts)
  deg8 = deg_flat.reshape(8, N_PAD)

  W1 = jnp.stack([W1_0, W1_1, W1_2, W1_3])
  W2 = jnp.stack([W2_0, W2_1, W2_2, W2_3])
  b1 = jnp.stack([b1_0, b1_1, b1_2, b1_3])
  b2 = jnp.stack([b2_0, b2_1, b2_2, b2_3])

  y1 = _tc_layer1(x_pad, deg8, W1)
  agg1 = _sc_aggregate(*y1, *srcs, *dsts, zrows)
  y2 = _tc_layer2(*agg1, deg8, b1, W2)
  agg2 = _sc_aggregate(*y2, *srcs, *dsts, zrows)
  out = _tc_final(*agg2, deg8, b2, Wc, bc.reshape(1, D_OUT))
  return out[:N]


# speedup vs baseline: 1.2087x; 1.2087x over previous
import functools

import jax
import jax.numpy as jnp
from jax import lax
from jax.experimental import pallas as pl
from jax.experimental.pallas import tpu as pltpu
from jax.experimental.pallas import tpu_sc as plsc

N = 10000
D = 128
D_OUT = 16
E = 80000
R = 4

NC = 2
NS = 16
LANES = 16

N_PAD = 10240
E_PAD = 81920
JUNK = 10200
CH = 128
ROWS_PER_TILE = N_PAD // NS
CHUNKS_PER_TILE = E_PAD // NS // CH
DEG_TOTAL = 4 * N_PAD
DEG_PER_TILE = DEG_TOTAL // NS

_sc_mesh = plsc.VectorSubcoreMesh(
    core_axis_name="c", subcore_axis_name="s", num_cores=NC, num_subcores=NS)


CH_E = 4096
N_CHUNKS_E = E_PAD // CH_E
SUB = N_PAD // NS


def _sc_degrees_body(s0, s1, s2, s3, d0, d1, d2, d3, deg_out,
                     idxbuf, subhist, histbuf):
  c = lax.axis_index("c")
  s = lax.axis_index("s")
  base = s * SUB
  lane_off = lax.broadcasted_iota(jnp.int32, (LANES,), 0) * SUB
  ones = jnp.ones((LANES,), jnp.float32)
  edge_arrays = ((s0, d0), (s1, d1), (s2, d2), (s3, d3))

  for rel in range(R):

    @pl.when(c == rel // 2)
    def _():
      for side in range(2):
        idx_hbm = edge_arrays[rel][side]
        a = 2 * rel + side

        def zero_blk(i, carry):
          subhist[pl.ds(i * LANES, LANES)] = jnp.zeros((LANES,), jnp.float32)
          return carry

        lax.fori_loop(0, (LANES * SUB) // LANES, zero_blk, 0)

        def chunk(k, carry):
          pltpu.sync_copy(idx_hbm.at[pl.ds(k * CH_E, CH_E)], idxbuf)

          def vec(j, carry2):
            idx = idxbuf[pl.ds(j * LANES, LANES)]
            rel_idx = idx - base
            m = (rel_idx >= 0) & (rel_idx < SUB)
            rel_c = jnp.clip(rel_idx, 0, SUB - 1)
            plsc.addupdate_scatter(subhist, [lane_off + rel_c], ones, mask=m)
            return carry2

          lax.fori_loop(0, CH_E // LANES, vec, 0)
          return carry

        lax.fori_loop(0, N_CHUNKS_E, chunk, 0)

        def reduce_blk(i, carry):
          acc = jnp.zeros((LANES,), jnp.float32)
          for lane in range(LANES):
            acc = acc + subhist[pl.ds(lane * SUB + i * LANES, LANES)]
          histbuf[pl.ds(i * LANES, LANES)] = acc
          return carry

        lax.fori_loop(0, SUB // LANES, reduce_blk, 0)
        pltpu.sync_copy(histbuf, deg_out.at[pl.ds(a * N_PAD + base, SUB)])


_sc_degrees = pl.kernel(
    _sc_degrees_body,
    out_type=jax.ShapeDtypeStruct((8 * N_PAD,), jnp.float32),
    mesh=_sc_mesh,
    compiler_params=pltpu.CompilerParams(needs_layout_passes=False),
    scratch_types=[
        pltpu.VMEM((CH_E,), jnp.int32),
        pltpu.VMEM((LANES * SUB,), jnp.float32),
        pltpu.VMEM((SUB,), jnp.float32),
    ],
)


def _sc_aggregate_body(y0, y1, y2, y3, s0, s1, s2, s3, d0, d1, d2, d3, zrows,
                       a0, a1, a2, a3, shacc,
                       idx_s0, idx_s1, idx_d0, idx_d1, rows0, rows1,
                       sem0, sem1):
  c = lax.axis_index("c")
  s = lax.axis_index("s")
  ys = (y0, y1, y2, y3)
  srcs = (s0, s1, s2, s3)
  dsts = (d0, d1, d2, d3)
  aggs = (a0, a1, a2, a3)
  idx_s = (idx_s0, idx_s1)
  idx_d = (idx_d0, idx_d1)
  rows = (rows0, rows1)
  sems = (sem0, sem1)
  row_sl = pl.ds(s * ROWS_PER_TILE, ROWS_PER_TILE)

  for rel in range(R):

    @pl.when(c == rel // 2)
    def _():
      y = ys[rel]
      src = srcs[rel]
      dst = dsts[rel]

      pltpu.sync_copy(zrows.at[row_sl], shacc.at[row_sl])
      plsc.subcore_barrier()

      def load_idx(k, p):
        off = s * (E_PAD // NS) + k * CH
        pltpu.sync_copy(src.at[pl.ds(off, CH)], idx_s[p])
        pltpu.sync_copy(dst.at[pl.ds(off, CH)], idx_d[p])

      load_idx(0, 0)
      handle = pltpu.async_copy(y.at[idx_s[0]], rows[0], sems[0])
      for k in range(CHUNKS_PER_TILE):
        p = k % 2
        if k + 1 < CHUNKS_PER_TILE:
          q = (k + 1) % 2
          load_idx(k + 1, q)
          next_handle = pltpu.async_copy(y.at[idx_s[q]], rows[q], sems[q])
        handle.wait()
        pltpu.sync_copy(rows[p], shacc.at[idx_d[p]], add=True)
        if k + 1 < CHUNKS_PER_TILE:
          handle = next_handle

      plsc.subcore_barrier()
      pltpu.sync_copy(shacc.at[row_sl], aggs[rel].at[row_sl])
      plsc.subcore_barrier()


_sc_aggregate = pl.kernel(
    _sc_aggregate_body,
    out_type=[jax.ShapeDtypeStruct((N_PAD, D), jnp.float32)] * R,
    mesh=_sc_mesh,
    compiler_params=pltpu.CompilerParams(use_tc_tiling_on_sc=False),
    scratch_types=[
        pltpu.VMEM_SHARED((N_PAD, D), jnp.float32),
        pltpu.VMEM((CH,), jnp.int32),
        pltpu.VMEM((CH,), jnp.int32),
        pltpu.VMEM((CH,), jnp.int32),
        pltpu.VMEM((CH,), jnp.int32),
        pltpu.VMEM((CH, D), jnp.float32),
        pltpu.VMEM((CH, D), jnp.float32),
        pltpu.SemaphoreType.DMA,
        pltpu.SemaphoreType.DMA,
    ],
)


_BLK = 512
_GRID = N_PAD // _BLK


def _norm(deg_row):
  return lax.rsqrt(jnp.maximum(deg_row, 1.0))


def _tc_layer1_body(x_ref, deg_ref, w_ref, y0, y1, y2, y3):
  xb = x_ref[...]
  outs = (y0, y1, y2, y3)
  for r in range(R):
    ns = _norm(deg_ref[2 * r])
    outs[r][...] = jnp.dot(xb * ns[:, None], w_ref[r],
                           preferred_element_type=jnp.float32)


_tc_layer1 = pl.pallas_call(
    _tc_layer1_body,
    grid=(_GRID,),
    in_specs=[
        pl.BlockSpec((_BLK, D), lambda i: (i, 0)),
        pl.BlockSpec((8, _BLK), lambda i: (0, i)),
        pl.BlockSpec((R, D, D), lambda i: (0, 0, 0)),
    ],
    out_specs=[pl.BlockSpec((_BLK, D), lambda i: (i, 0))] * R,
    out_shape=[jax.ShapeDtypeStruct((N_PAD, D), jnp.float32)] * R,
)


def _tc_layer2_body(a0, a1, a2, a3, deg_ref, b_ref, w_ref, y0, y1, y2, y3):
  aggs = (a0, a1, a2, a3)
  h = jnp.zeros((_BLK, D), jnp.float32)
  for r in range(R):
    nd = _norm(deg_ref[2 * r + 1])
    h = h + aggs[r][...] * nd[:, None] + b_ref[r][None, :]
  h = jnp.maximum(h, 0.0)
  outs = (y0, y1, y2, y3)
  for r in range(R):
    ns = _norm(deg_ref[2 * r])
    outs[r][...] = jnp.dot(h * ns[:, None], w_ref[r],
                           preferred_element_type=jnp.float32)


_tc_layer2 = pl.pallas_call(
    _tc_layer2_body,
    grid=(_GRID,),
    in_specs=[pl.BlockSpec((_BLK, D), lambda i: (i, 0))] * R + [
        pl.BlockSpec((8, _BLK), lambda i: (0, i)),
        pl.BlockSpec((R, D), lambda i: (0, 0)),
        pl.BlockSpec((R, D, D), lambda i: (0, 0, 0)),
    ],
    out_specs=[pl.BlockSpec((_BLK, D), lambda i: (i, 0))] * R,
    out_shape=[jax.ShapeDtypeStruct((N_PAD, D), jnp.float32)] * R,
)


def _tc_final_body(a0, a1, a2, a3, deg_ref, b_ref, wc_ref, bc_ref, out_ref):
  aggs = (a0, a1, a2, a3)
  h = jnp.zeros((_BLK, D), jnp.float32)
  for r in range(R):
    nd = _norm(deg_ref[2 * r + 1])
    h = h + aggs[r][...] * nd[:, None] + b_ref[r][None, :]
  h = jnp.maximum(h, 0.0)
  out_ref[...] = jnp.dot(h, wc_ref[...],
                         preferred_element_type=jnp.float32) + bc_ref[0][None, :]


_tc_final = pl.pallas_call(
    _tc_final_body,
    grid=(_GRID,),
    in_specs=[pl.BlockSpec((_BLK, D), lambda i: (i, 0))] * R + [
        pl.BlockSpec((8, _BLK), lambda i: (0, i)),
        pl.BlockSpec((R, D), lambda i: (0, 0)),
        pl.BlockSpec((D, D_OUT), lambda i: (0, 0)),
        pl.BlockSpec((1, D_OUT), lambda i: (0, 0)),
    ],
    out_specs=pl.BlockSpec((_BLK, D_OUT), lambda i: (i, 0)),
    out_shape=jax.ShapeDtypeStruct((N_PAD, D_OUT), jnp.float32),
)


@jax.jit
def kernel(x, W1_0, b1_0, W1_1, b1_1, W1_2, b1_2, W1_3, b1_3,
           W2_0, b2_0, W2_1, b2_1, W2_2, b2_2, W2_3, b2_3,
           Wc, bc, edge_index0, edge_index1, edge_index2, edge_index3):
  edges = (edge_index0, edge_index1, edge_index2, edge_index3)
  pad = jnp.full((E_PAD - E,), JUNK, dtype=jnp.int32)
  srcs = [jnp.concatenate([e[0], pad]) for e in edges]
  dsts = [jnp.concatenate([e[1], pad]) for e in edges]

  x_pad = jnp.zeros((N_PAD, D), jnp.float32).at[:N].set(x)
  zrows = jnp.zeros((N_PAD, D), jnp.float32)

  deg_flat = _sc_degrees(*srcs, *dsts)
  deg8 = deg_flat.reshape(8, N_PAD)

  W1 = jnp.stack([W1_0, W1_1, W1_2, W1_3])
  W2 = jnp.stack([W2_0, W2_1, W2_2, W2_3])
  b1 = jnp.stack([b1_0, b1_1, b1_2, b1_3])
  b2 = jnp.stack([b2_0, b2_1, b2_2, b2_3])

  y1 = _tc_layer1(x_pad, deg8, W1)
  agg1 = _sc_aggregate(*y1, *srcs, *dsts, zrows)
  y2 = _tc_layer2(*agg1, deg8, b1, W2)
  agg2 = _sc_aggregate(*y2, *srcs, *dsts, zrows)
  out = _tc_final(*agg2, deg8, b2, Wc, bc.reshape(1, D_OUT))
  return out[:N]


# speedup vs baseline: 1.6149x; 1.3361x over previous
import functools

import jax
import jax.numpy as jnp
from jax import lax
from jax.experimental import pallas as pl
from jax.experimental.pallas import tpu as pltpu
from jax.experimental.pallas import tpu_sc as plsc

N = 10000
D = 128
D_OUT = 16
E = 80000
R = 4

NC = 2
NS = 16
LANES = 16

N_PAD = 10240
E_PAD = 81920
JUNK = 10200
CH = 128
ROWS_PER_TILE = N_PAD // NS
CHUNKS_PER_TILE = E_PAD // NS // CH
DEG_TOTAL = 4 * N_PAD
DEG_PER_TILE = DEG_TOTAL // NS

_sc_mesh = plsc.VectorSubcoreMesh(
    core_axis_name="c", subcore_axis_name="s", num_cores=NC, num_subcores=NS)


CH_D = 512
N_CH_D = E_PAD // NS // CH_D
SUB = N_PAD // NS


def _sc_degrees_body(s0, s1, s2, s3, d0, d1, d2, d3, deg_out,
                     h0, h1, h2, h3, idxb0, idxb1, ones, zbuf,
                     sem0, sem1):
  c = lax.axis_index("c")
  s = lax.axis_index("s")
  hists = (h0, h1, h2, h3)
  idxb = (idxb0, idxb1)
  sems = (sem0, sem1)
  srcs = (s0, s1, s2, s3)
  dsts = (d0, d1, d2, d3)
  tile_sl = pl.ds(s * SUB, SUB)

  def init_ones(i, carry):
    ones[pl.ds(i * LANES, LANES)] = jnp.ones((LANES,), jnp.float32)
    return carry

  lax.fori_loop(0, CH_D // LANES, init_ones, 0)

  def init_zeros(i, carry):
    zbuf[pl.ds(i * LANES, LANES)] = jnp.zeros((LANES,), jnp.float32)
    return carry

  lax.fori_loop(0, SUB // LANES, init_zeros, 0)

  for a in range(4):
    pltpu.sync_copy(zbuf, hists[a].at[tile_sl])
  plsc.subcore_barrier()

  for half in range(NC):

    @pl.when(c == half)
    def _():
      for a_local in range(4):
        rel = 2 * half + a_local // 2
        arr = (srcs if a_local % 2 == 0 else dsts)[rel]
        hist = hists[a_local]

        def start_load(k, p):
          off = s * (E_PAD // NS) + k * CH_D
          return pltpu.async_copy(arr.at[pl.ds(off, CH_D)], idxb[p], sems[p])

        handle = start_load(0, 0)
        for k in range(N_CH_D):
          p = k % 2
          if k + 1 < N_CH_D:
            next_handle = start_load(k + 1, (k + 1) % 2)
          handle.wait()
          pltpu.sync_copy(ones, hist.at[idxb[p]], add=True)
          if k + 1 < N_CH_D:
            handle = next_handle

  plsc.subcore_barrier()
  for half in range(NC):

    @pl.when(c == half)
    def _():
      for a_local in range(4):
        a = 4 * half + a_local
        pltpu.sync_copy(hists[a_local].at[tile_sl],
                        deg_out.at[pl.ds(a * N_PAD + s * SUB, SUB)])


_sc_degrees = pl.kernel(
    _sc_degrees_body,
    out_type=jax.ShapeDtypeStruct((8 * N_PAD,), jnp.float32),
    mesh=_sc_mesh,
    compiler_params=pltpu.CompilerParams(use_tc_tiling_on_sc=False),
    scratch_types=[
        pltpu.VMEM_SHARED((N_PAD,), jnp.float32),
        pltpu.VMEM_SHARED((N_PAD,), jnp.float32),
        pltpu.VMEM_SHARED((N_PAD,), jnp.float32),
        pltpu.VMEM_SHARED((N_PAD,), jnp.float32),
        pltpu.VMEM((CH_D,), jnp.int32),
        pltpu.VMEM((CH_D,), jnp.int32),
        pltpu.VMEM((CH_D,), jnp.float32),
        pltpu.VMEM((SUB,), jnp.float32),
        pltpu.SemaphoreType.DMA,
        pltpu.SemaphoreType.DMA,
    ],
)


def _sc_aggregate_body(y0, y1, y2, y3, s0, s1, s2, s3, d0, d1, d2, d3, zrows,
                       a0, a1, a2, a3, shacc,
                       idx_s0, idx_s1, idx_d0, idx_d1, rows0, rows1,
                       sem0, sem1):
  c = lax.axis_index("c")
  s = lax.axis_index("s")
  ys = (y0, y1, y2, y3)
  srcs = (s0, s1, s2, s3)
  dsts = (d0, d1, d2, d3)
  aggs = (a0, a1, a2, a3)
  idx_s = (idx_s0, idx_s1)
  idx_d = (idx_d0, idx_d1)
  rows = (rows0, rows1)
  sems = (sem0, sem1)
  row_sl = pl.ds(s * ROWS_PER_TILE, ROWS_PER_TILE)

  for rel in range(R):

    @pl.when(c == rel // 2)
    def _():
      y = ys[rel]
      src = srcs[rel]
      dst = dsts[rel]

      pltpu.sync_copy(zrows.at[row_sl], shacc.at[row_sl])
      plsc.subcore_barrier()

      def load_idx(k, p):
        off = s * (E_PAD // NS) + k * CH
        pltpu.sync_copy(src.at[pl.ds(off, CH)], idx_s[p])
        pltpu.sync_copy(dst.at[pl.ds(off, CH)], idx_d[p])

      load_idx(0, 0)
      handle = pltpu.async_copy(y.at[idx_s[0]], rows[0], sems[0])
      for k in range(CHUNKS_PER_TILE):
        p = k % 2
        if k + 1 < CHUNKS_PER_TILE:
          q = (k + 1) % 2
          load_idx(k + 1, q)
          next_handle = pltpu.async_copy(y.at[idx_s[q]], rows[q], sems[q])
        handle.wait()
        pltpu.sync_copy(rows[p], shacc.at[idx_d[p]], add=True)
        if k + 1 < CHUNKS_PER_TILE:
          handle = next_handle

      plsc.subcore_barrier()
      pltpu.sync_copy(shacc.at[row_sl], aggs[rel].at[row_sl])
      plsc.subcore_barrier()


_sc_aggregate = pl.kernel(
    _sc_aggregate_body,
    out_type=[jax.ShapeDtypeStruct((N_PAD, D), jnp.float32)] * R,
    mesh=_sc_mesh,
    compiler_params=pltpu.CompilerParams(use_tc_tiling_on_sc=False),
    scratch_types=[
        pltpu.VMEM_SHARED((N_PAD, D), jnp.float32),
        pltpu.VMEM((CH,), jnp.int32),
        pltpu.VMEM((CH,), jnp.int32),
        pltpu.VMEM((CH,), jnp.int32),
        pltpu.VMEM((CH,), jnp.int32),
        pltpu.VMEM((CH, D), jnp.float32),
        pltpu.VMEM((CH, D), jnp.float32),
        pltpu.SemaphoreType.DMA,
        pltpu.SemaphoreType.DMA,
    ],
)


_BLK = 512
_GRID = N_PAD // _BLK


def _norm(deg_row):
  return lax.rsqrt(jnp.maximum(deg_row, 1.0))


def _tc_layer1_body(x_ref, deg_ref, w_ref, y0, y1, y2, y3):
  xb = x_ref[...]
  outs = (y0, y1, y2, y3)
  for r in range(R):
    ns = _norm(deg_ref[2 * r])
    outs[r][...] = jnp.dot(xb * ns[:, None], w_ref[r],
                           preferred_element_type=jnp.float32)


_tc_layer1 = pl.pallas_call(
    _tc_layer1_body,
    grid=(_GRID,),
    in_specs=[
        pl.BlockSpec((_BLK, D), lambda i: (i, 0)),
        pl.BlockSpec((8, _BLK), lambda i: (0, i)),
        pl.BlockSpec((R, D, D), lambda i: (0, 0, 0)),
    ],
    out_specs=[pl.BlockSpec((_BLK, D), lambda i: (i, 0))] * R,
    out_shape=[jax.ShapeDtypeStruct((N_PAD, D), jnp.float32)] * R,
)


def _tc_layer2_body(a0, a1, a2, a3, deg_ref, b_ref, w_ref, y0, y1, y2, y3):
  aggs = (a0, a1, a2, a3)
  h = jnp.zeros((_BLK, D), jnp.float32)
  for r in range(R):
    nd = _norm(deg_ref[2 * r + 1])
    h = h + aggs[r][...] * nd[:, None] + b_ref[r][None, :]
  h = jnp.maximum(h, 0.0)
  outs = (y0, y1, y2, y3)
  for r in range(R):
    ns = _norm(deg_ref[2 * r])
    outs[r][...] = jnp.dot(h * ns[:, None], w_ref[r],
                           preferred_element_type=jnp.float32)


_tc_layer2 = pl.pallas_call(
    _tc_layer2_body,
    grid=(_GRID,),
    in_specs=[pl.BlockSpec((_BLK, D), lambda i: (i, 0))] * R + [
        pl.BlockSpec((8, _BLK), lambda i: (0, i)),
        pl.BlockSpec((R, D), lambda i: (0, 0)),
        pl.BlockSpec((R, D, D), lambda i: (0, 0, 0)),
    ],
    out_specs=[pl.BlockSpec((_BLK, D), lambda i: (i, 0))] * R,
    out_shape=[jax.ShapeDtypeStruct((N_PAD, D), jnp.float32)] * R,
)


def _tc_final_body(a0, a1, a2, a3, deg_ref, b_ref, wc_ref, bc_ref, out_ref):
  aggs = (a0, a1, a2, a3)
  h = jnp.zeros((_BLK, D), jnp.float32)
  for r in range(R):
    nd = _norm(deg_ref[2 * r + 1])
    h = h + aggs[r][...] * nd[:, None] + b_ref[r][None, :]
  h = jnp.maximum(h, 0.0)
  out_ref[...] = jnp.dot(h, wc_ref[...],
                         preferred_element_type=jnp.float32) + bc_ref[0][None, :]


_tc_final = pl.pallas_call(
    _tc_final_body,
    grid=(_GRID,),
    in_specs=[pl.BlockSpec((_BLK, D), lambda i: (i, 0))] * R + [
        pl.BlockSpec((8, _BLK), lambda i: (0, i)),
        pl.BlockSpec((R, D), lambda i: (0, 0)),
        pl.BlockSpec((D, D_OUT), lambda i: (0, 0)),
        pl.BlockSpec((1, D_OUT), lambda i: (0, 0)),
    ],
    out_specs=pl.BlockSpec((_BLK, D_OUT), lambda i: (i, 0)),
    out_shape=jax.ShapeDtypeStruct((N_PAD, D_OUT), jnp.float32),
)


@jax.jit
def kernel(x, W1_0, b1_0, W1_1, b1_1, W1_2, b1_2, W1_3, b1_3,
           W2_0, b2_0, W2_1, b2_1, W2_2, b2_2, W2_3, b2_3,
           Wc, bc, edge_index0, edge_index1, edge_index2, edge_index3):
  edges = (edge_index0, edge_index1, edge_index2, edge_index3)
  pad = jnp.full((E_PAD - E,), JUNK, dtype=jnp.int32)
  srcs = [jnp.concatenate([e[0], pad]) for e in edges]
  dsts = [jnp.concatenate([e[1], pad]) for e in edges]

  x_pad = jnp.zeros((N_PAD, D), jnp.float32).at[:N].set(x)
  zrows = jnp.zeros((N_PAD, D), jnp.float32)

  deg_flat = _sc_degrees(*srcs, *dsts)
  deg8 = deg_flat.reshape(8, N_PAD)

  W1 = jnp.stack([W1_0, W1_1, W1_2, W1_3])
  W2 = jnp.stack([W2_0, W2_1, W2_2, W2_3])
  b1 = jnp.stack([b1_0, b1_1, b1_2, b1_3])
  b2 = jnp.stack([b2_0, b2_1, b2_2, b2_3])

  y1 = _tc_layer1(x_pad, deg8, W1)
  agg1 = _sc_aggregate(*y1, *srcs, *dsts, zrows)
  y2 = _tc_layer2(*agg1, deg8, b1, W2)
  agg2 = _sc_aggregate(*y2, *srcs, *dsts, zrows)
  out = _tc_final(*agg2, deg8, b2, Wc, bc.reshape(1, D_OUT))
  return out[:N]


# speedup vs baseline: 1.7264x; 1.0691x over previous
import functools

import jax
import jax.numpy as jnp
from jax import lax
from jax.experimental import pallas as pl
from jax.experimental.pallas import tpu as pltpu
from jax.experimental.pallas import tpu_sc as plsc

N = 10000
D = 128
D_OUT = 16
E = 80000
R = 4

NC = 2
NS = 16
LANES = 16

N_PAD = 10240
E_PAD = 81920
JUNK = 10200
CH = 128
ROWS_PER_TILE = N_PAD // NS
CHUNKS_PER_TILE = E_PAD // NS // CH
DEG_TOTAL = 4 * N_PAD
DEG_PER_TILE = DEG_TOTAL // NS

_sc_mesh = plsc.VectorSubcoreMesh(
    core_axis_name="c", subcore_axis_name="s", num_cores=NC, num_subcores=NS)


CH_D = 512
N_CH_D = E_PAD // NS // CH_D
SUB = N_PAD // NS


def _sc_degrees_body(s0, s1, s2, s3, d0, d1, d2, d3, deg_out,
                     h0, h1, h2, h3, idxb0, idxb1, ones, zbuf,
                     sem0, sem1):
  c = lax.axis_index("c")
  s = lax.axis_index("s")
  hists = (h0, h1, h2, h3)
  idxb = (idxb0, idxb1)
  sems = (sem0, sem1)
  srcs = (s0, s1, s2, s3)
  dsts = (d0, d1, d2, d3)
  tile_sl = pl.ds(s * SUB, SUB)

  def init_ones(i, carry):
    ones[pl.ds(i * LANES, LANES)] = jnp.ones((LANES,), jnp.float32)
    return carry

  lax.fori_loop(0, CH_D // LANES, init_ones, 0)

  def init_zeros(i, carry):
    zbuf[pl.ds(i * LANES, LANES)] = jnp.zeros((LANES,), jnp.float32)
    return carry

  lax.fori_loop(0, SUB // LANES, init_zeros, 0)

  for a in range(4):
    pltpu.sync_copy(zbuf, hists[a].at[tile_sl])
  plsc.subcore_barrier()

  for half in range(NC):

    @pl.when(c == half)
    def _():
      for a_local in range(4):
        rel = 2 * half + a_local // 2
        arr = (srcs if a_local % 2 == 0 else dsts)[rel]
        hist = hists[a_local]

        def start_load(k, p):
          off = s * (E_PAD // NS) + k * CH_D
          return pltpu.async_copy(arr.at[pl.ds(off, CH_D)], idxb[p], sems[p])

        handle = start_load(0, 0)
        for k in range(N_CH_D):
          p = k % 2
          if k + 1 < N_CH_D:
            next_handle = start_load(k + 1, (k + 1) % 2)
          handle.wait()
          pltpu.sync_copy(ones, hist.at[idxb[p]], add=True)
          if k + 1 < N_CH_D:
            handle = next_handle

  plsc.subcore_barrier()
  for half in range(NC):

    @pl.when(c == half)
    def _():
      for a_local in range(4):
        a = 4 * half + a_local
        pltpu.sync_copy(hists[a_local].at[tile_sl],
                        deg_out.at[pl.ds(a * N_PAD + s * SUB, SUB)])


_sc_degrees = pl.kernel(
    _sc_degrees_body,
    out_type=jax.ShapeDtypeStruct((8 * N_PAD,), jnp.float32),
    mesh=_sc_mesh,
    compiler_params=pltpu.CompilerParams(use_tc_tiling_on_sc=False),
    scratch_types=[
        pltpu.VMEM_SHARED((N_PAD,), jnp.float32),
        pltpu.VMEM_SHARED((N_PAD,), jnp.float32),
        pltpu.VMEM_SHARED((N_PAD,), jnp.float32),
        pltpu.VMEM_SHARED((N_PAD,), jnp.float32),
        pltpu.VMEM((CH_D,), jnp.int32),
        pltpu.VMEM((CH_D,), jnp.int32),
        pltpu.VMEM((CH_D,), jnp.float32),
        pltpu.VMEM((SUB,), jnp.float32),
        pltpu.SemaphoreType.DMA,
        pltpu.SemaphoreType.DMA,
    ],
)


EPT = E_PAD // NS


def _sc_aggregate_body(y0, y1, y2, y3, s0, s1, s2, s3, d0, d1, d2, d3, zrows,
                       a0, a1, a2, a3, shacc,
                       idx_all_s, idx_all_d, rows0, rows1,
                       sem0, sem1, sem_is, sem_id):
  c = lax.axis_index("c")
  s = lax.axis_index("s")
  ys = (y0, y1, y2, y3)
  srcs = (s0, s1, s2, s3)
  dsts = (d0, d1, d2, d3)
  aggs = (a0, a1, a2, a3)
  rows = (rows0, rows1)
  sems = (sem0, sem1)
  row_sl = pl.ds(s * ROWS_PER_TILE, ROWS_PER_TILE)

  for rel in range(R):

    @pl.when(c == rel // 2)
    def _():
      y = ys[rel]
      src = srcs[rel]
      dst = dsts[rel]

      hs = pltpu.async_copy(src.at[pl.ds(s * EPT, EPT)], idx_all_s, sem_is)
      hd = pltpu.async_copy(dst.at[pl.ds(s * EPT, EPT)], idx_all_d, sem_id)
      pltpu.sync_copy(zrows.at[row_sl], shacc.at[row_sl])
      hs.wait()
      hd.wait()
      plsc.subcore_barrier()

      def gather(k, p):
        return pltpu.async_copy(
            y.at[idx_all_s.at[pl.ds(k * CH, CH)]], rows[p], sems[p])

      handle = gather(0, 0)
      for k in range(CHUNKS_PER_TILE):
        p = k % 2
        if k + 1 < CHUNKS_PER_TILE:
          next_handle = gather(k + 1, (k + 1) % 2)
        handle.wait()
        pltpu.sync_copy(rows[p],
                        shacc.at[idx_all_d.at[pl.ds(k * CH, CH)]], add=True)
        if k + 1 < CHUNKS_PER_TILE:
          handle = next_handle

      plsc.subcore_barrier()
      pltpu.sync_copy(shacc.at[row_sl], aggs[rel].at[row_sl])
      plsc.subcore_barrier()


_sc_aggregate = pl.kernel(
    _sc_aggregate_body,
    out_type=[jax.ShapeDtypeStruct((N_PAD, D), jnp.float32)] * R,
    mesh=_sc_mesh,
    compiler_params=pltpu.CompilerParams(use_tc_tiling_on_sc=False),
    scratch_types=[
        pltpu.VMEM_SHARED((N_PAD, D), jnp.float32),
        pltpu.VMEM((EPT,), jnp.int32),
        pltpu.VMEM((EPT,), jnp.int32),
        pltpu.VMEM((CH, D), jnp.float32),
        pltpu.VMEM((CH, D), jnp.float32),
        pltpu.SemaphoreType.DMA,
        pltpu.SemaphoreType.DMA,
        pltpu.SemaphoreType.DMA,
        pltpu.SemaphoreType.DMA,
    ],
)


_BLK = 512
_GRID = N_PAD // _BLK


def _norm(deg_row):
  return lax.rsqrt(jnp.maximum(deg_row, 1.0))


def _tc_layer1_body(x_ref, deg_ref, w_ref, y0, y1, y2, y3):
  xb = x_ref[...]
  outs = (y0, y1, y2, y3)
  for r in range(R):
    ns = _norm(deg_ref[2 * r])
    outs[r][...] = jnp.dot(xb * ns[:, None], w_ref[r],
                           preferred_element_type=jnp.float32)


_tc_layer1 = pl.pallas_call(
    _tc_layer1_body,
    grid=(_GRID,),
    in_specs=[
        pl.BlockSpec((_BLK, D), lambda i: (i, 0)),
        pl.BlockSpec((8, _BLK), lambda i: (0, i)),
        pl.BlockSpec((R, D, D), lambda i: (0, 0, 0)),
    ],
    out_specs=[pl.BlockSpec((_BLK, D), lambda i: (i, 0))] * R,
    out_shape=[jax.ShapeDtypeStruct((N_PAD, D), jnp.float32)] * R,
)


def _tc_layer2_body(a0, a1, a2, a3, deg_ref, b_ref, w_ref, y0, y1, y2, y3):
  aggs = (a0, a1, a2, a3)
  h = jnp.zeros((_BLK, D), jnp.float32)
  for r in range(R):
    nd = _norm(deg_ref[2 * r + 1])
    h = h + aggs[r][...] * nd[:, None] + b_ref[r][None, :]
  h = jnp.maximum(h, 0.0)
  outs = (y0, y1, y2, y3)
  for r in range(R):
    ns = _norm(deg_ref[2 * r])
    outs[r][...] = jnp.dot(h * ns[:, None], w_ref[r],
                           preferred_element_type=jnp.float32)


_tc_layer2 = pl.pallas_call(
    _tc_layer2_body,
    grid=(_GRID,),
    in_specs=[pl.BlockSpec((_BLK, D), lambda i: (i, 0))] * R + [
        pl.BlockSpec((8, _BLK), lambda i: (0, i)),
        pl.BlockSpec((R, D), lambda i: (0, 0)),
        pl.BlockSpec((R, D, D), lambda i: (0, 0, 0)),
    ],
    out_specs=[pl.BlockSpec((_BLK, D), lambda i: (i, 0))] * R,
    out_shape=[jax.ShapeDtypeStruct((N_PAD, D), jnp.float32)] * R,
)


def _tc_final_body(a0, a1, a2, a3, deg_ref, b_ref, wc_ref, bc_ref, out_ref):
  aggs = (a0, a1, a2, a3)
  h = jnp.zeros((_BLK, D), jnp.float32)
  for r in range(R):
    nd = _norm(deg_ref[2 * r + 1])
    h = h + aggs[r][...] * nd[:, None] + b_ref[r][None, :]
  h = jnp.maximum(h, 0.0)
  out_ref[...] = jnp.dot(h, wc_ref[...],
                         preferred_element_type=jnp.float32) + bc_ref[0][None, :]


_tc_final = pl.pallas_call(
    _tc_final_body,
    grid=(_GRID,),
    in_specs=[pl.BlockSpec((_BLK, D), lambda i: (i, 0))] * R + [
        pl.BlockSpec((8, _BLK), lambda i: (0, i)),
        pl.BlockSpec((R, D), lambda i: (0, 0)),
        pl.BlockSpec((D, D_OUT), lambda i: (0, 0)),
        pl.BlockSpec((1, D_OUT), lambda i: (0, 0)),
    ],
    out_specs=pl.BlockSpec((_BLK, D_OUT), lambda i: (i, 0)),
    out_shape=jax.ShapeDtypeStruct((N_PAD, D_OUT), jnp.float32),
)


@jax.jit
def kernel(x, W1_0, b1_0, W1_1, b1_1, W1_2, b1_2, W1_3, b1_3,
           W2_0, b2_0, W2_1, b2_1, W2_2, b2_2, W2_3, b2_3,
           Wc, bc, edge_index0, edge_index1, edge_index2, edge_index3):
  edges = (edge_index0, edge_index1, edge_index2, edge_index3)
  pad = jnp.full((E_PAD - E,), JUNK, dtype=jnp.int32)
  srcs = [jnp.concatenate([e[0], pad]) for e in edges]
  dsts = [jnp.concatenate([e[1], pad]) for e in edges]

  x_pad = jnp.zeros((N_PAD, D), jnp.float32).at[:N].set(x)
  zrows = jnp.zeros((N_PAD, D), jnp.float32)

  deg_flat = _sc_degrees(*srcs, *dsts)
  deg8 = deg_flat.reshape(8, N_PAD)

  W1 = jnp.stack([W1_0, W1_1, W1_2, W1_3])
  W2 = jnp.stack([W2_0, W2_1, W2_2, W2_3])
  b1 = jnp.stack([b1_0, b1_1, b1_2, b1_3])
  b2 = jnp.stack([b2_0, b2_1, b2_2, b2_3])

  y1 = _tc_layer1(x_pad, deg8, W1)
  agg1 = _sc_aggregate(*y1, *srcs, *dsts, zrows)
  y2 = _tc_layer2(*agg1, deg8, b1, W2)
  agg2 = _sc_aggregate(*y2, *srcs, *dsts, zrows)
  out = _tc_final(*agg2, deg8, b2, Wc, bc.reshape(1, D_OUT))
  return out[:N]
